# baseline (speedup 1.0000x reference)
import jax
import jax.numpy as jnp
from jax.experimental import pallas as pl
from jax.experimental.pallas import tpu as pltpu

_B, _N, _C = 128, 25, 128
_J = 25
_HID = 32
_K = 8
_E = _J * _J
_U = 16
_G = _B // _U


def _edge_attn_body(spd_ref, src_ref, w1_ref, a_ref, w2_ref, out_ref,
                    dt_ref, w1f_ref, cpos_ref, cneg_ref):
    @pl.when(pl.program_id(0) == 0)
    def _prep():
        spdt = spd_ref[...].T
        n_iota = jax.lax.broadcasted_iota(jnp.int32, (_N, _E), 0)
        dt = jnp.zeros((_N, _E), dtype=jnp.float32)
        for k in range(_K - 1):
            ends_k = spdt[k][None, :]
            heads_k = spdt[k][None, :]
            dt = dt + (ends_k == n_iota).astype(jnp.float32)
            dt = dt - (heads_k == n_iota).astype(jnp.float32)
        dt_ref[...] = dt.astype(jnp.bfloat16)

        alpha = a_ref[0, 0]
        w2col = w2_ref[...].T
        w1f_ref[...] = (w1_ref[...] * w2col).astype(jnp.bfloat16)
        cpos1 = jnp.where(w2col >= 0, 1.0, alpha)
        cneg1 = jnp.where(w2col >= 0, alpha, 1.0)
        cpos_ref[...] = jnp.concatenate([cpos1] * _U, axis=0)
        cneg_ref[...] = jnp.concatenate([cneg1] * _U, axis=0)

    dt = dt_ref[...]
    w1f = w1f_ref[...]
    cpos = cpos_ref[...]
    cneg = cneg_ref[...]

    pts = [
        jnp.dot(w1f, src_ref[j].T.astype(jnp.bfloat16),
                preferred_element_type=jnp.float32)
        for j in range(_U)
    ]
    pcat = jnp.concatenate(pts, axis=0).astype(jnp.bfloat16)
    hcat = jnp.dot(pcat, dt, preferred_element_type=jnp.float32)
    g = cpos * jnp.maximum(hcat, 0.0) + cneg * jnp.minimum(hcat, 0.0)
    out_ref[...] = jnp.sum(g.reshape(_U, _HID, _E), axis=1)


def kernel(src, s_SPD, W1, a, W2):
    spd = s_SPD.reshape(_E, _K)
    a2 = a.reshape(1, 1)
    out = pl.pallas_call(
        _edge_attn_body,
        grid=(_G,),
        in_specs=[
            pl.BlockSpec((_E, _K), lambda i: (0, 0)),
            pl.BlockSpec((_U, _N, _C), lambda i: (i, 0, 0)),
            pl.BlockSpec((_HID, _C), lambda i: (0, 0)),
            pl.BlockSpec((1, 1), lambda i: (0, 0)),
            pl.BlockSpec((1, _HID), lambda i: (0, 0)),
        ],
        out_specs=pl.BlockSpec((_U, _E), lambda i: (i, 0)),
        out_shape=jax.ShapeDtypeStruct((_B, _E), jnp.float32),
        scratch_shapes=[
            pltpu.VMEM((_N, _E), jnp.bfloat16),
            pltpu.VMEM((_HID, _C), jnp.bfloat16),
            pltpu.VMEM((_U * _HID, 1), jnp.float32),
            pltpu.VMEM((_U * _HID, 1), jnp.float32),
        ],
    )(spd, src, W1, a2, W2)
    return out.reshape(_B, _J, _J, 1)


# speedup vs baseline: 1.1029x; 1.1029x over previous
import jax
import jax.numpy as jnp
from jax.experimental import pallas as pl
from jax.experimental.pallas import tpu as pltpu

_B, _N, _C = 128, 25, 128
_J = 25
_HID = 32
_K = 8
_E = _J * _J
_U = 64
_G = _B // _U


def _edge_attn_body(spd_ref, src_ref, w1_ref, a_ref, w2_ref, out_ref,
                    dt_ref, w1f_ref, cpos_ref, cneg_ref):
    @pl.when(pl.program_id(0) == 0)
    def _prep():
        spdt = spd_ref[...].T
        n_iota = jax.lax.broadcasted_iota(jnp.int32, (_N, _E), 0)
        dt = jnp.zeros((_N, _E), dtype=jnp.float32)
        for k in range(_K - 1):
            ends_k = spdt[k][None, :]
            heads_k = spdt[k][None, :]
            dt = dt + (ends_k == n_iota).astype(jnp.float32)
            dt = dt - (heads_k == n_iota).astype(jnp.float32)
        dt_ref[...] = dt.astype(jnp.bfloat16)

        alpha = a_ref[0, 0]
        w2col = w2_ref[...].T
        w1f_ref[...] = (w1_ref[...] * w2col).astype(jnp.bfloat16)
        cpos1 = jnp.where(w2col >= 0, 1.0, alpha)
        cneg1 = jnp.where(w2col >= 0, alpha, 1.0)
        cpos_ref[...] = jnp.concatenate([cpos1] * _U, axis=0)
        cneg_ref[...] = jnp.concatenate([cneg1] * _U, axis=0)

    dt = dt_ref[...]
    w1f = w1f_ref[...]
    cpos = cpos_ref[...]
    cneg = cneg_ref[...]

    pts = [
        jnp.dot(w1f, src_ref[j].T.astype(jnp.bfloat16),
                preferred_element_type=jnp.float32)
        for j in range(_U)
    ]
    pcat = jnp.concatenate(pts, axis=0).astype(jnp.bfloat16)
    hcat = jnp.dot(pcat, dt, preferred_element_type=jnp.float32)
    g = cpos * jnp.maximum(hcat, 0.0) + cneg * jnp.minimum(hcat, 0.0)
    out_ref[...] = jnp.sum(g.reshape(_U, _HID, _E), axis=1)


def kernel(src, s_SPD, W1, a, W2):
    spd = s_SPD.reshape(_E, _K)
    a2 = a.reshape(1, 1)
    out = pl.pallas_call(
        _edge_attn_body,
        grid=(_G,),
        in_specs=[
            pl.BlockSpec((_E, _K), lambda i: (0, 0)),
            pl.BlockSpec((_U, _N, _C), lambda i: (i, 0, 0)),
            pl.BlockSpec((_HID, _C), lambda i: (0, 0)),
            pl.BlockSpec((1, 1), lambda i: (0, 0)),
            pl.BlockSpec((1, _HID), lambda i: (0, 0)),
        ],
        out_specs=pl.BlockSpec((_U, _E), lambda i: (i, 0)),
        out_shape=jax.ShapeDtypeStruct((_B, _E), jnp.float32),
        scratch_shapes=[
            pltpu.VMEM((_N, _E), jnp.bfloat16),
            pltpu.VMEM((_HID, _C), jnp.bfloat16),
            pltpu.VMEM((_U * _HID, 1), jnp.float32),
            pltpu.VMEM((_U * _HID, 1), jnp.float32),
        ],
    )(spd, src, W1, a2, W2)
    return out.reshape(_B, _J, _J, 1)


# speedup vs baseline: 1.1389x; 1.0326x over previous
import jax
import jax.numpy as jnp
from jax.experimental import pallas as pl
from jax.experimental.pallas import tpu as pltpu

_B, _N, _C = 128, 25, 128
_J = 25
_HID = 32
_K = 8
_E = _J * _J
_U = 32
_G = _B // _U


def _edge_attn_body(spd_ref, src_ref, w1_ref, a_ref, w2_ref, out_ref,
                    dt_ref, w1f_ref, sblk_ref):
    @pl.when(pl.program_id(0) == 0)
    def _prep():
        spdt = spd_ref[...].T
        n_iota = jax.lax.broadcasted_iota(jnp.int32, (_N, _E), 0)
        dt = jnp.zeros((_N, _E), dtype=jnp.float32)
        for k in range(_K - 1):
            ends_k = spdt[k][None, :]
            heads_k = spdt[k][None, :]
            dt = dt + (ends_k == n_iota).astype(jnp.float32)
            dt = dt - (heads_k == n_iota).astype(jnp.float32)
        dt_ref[...] = dt.astype(jnp.bfloat16)

        alpha = a_ref[0, 0]
        w2row = w2_ref[...]
        w2col = w2row.T
        w1f_ref[...] = (w1_ref[...] * w2col).astype(jnp.bfloat16)
        c2 = (1.0 - alpha) * 0.5
        s_lane = jnp.where(w2row >= 0, c2, -c2)
        s_tile = jnp.concatenate([s_lane] * _U, axis=1)
        row_i = jax.lax.broadcasted_iota(jnp.int32, (_U, _U * _HID), 0)
        col_i = jax.lax.broadcasted_iota(jnp.int32, (_U, _U * _HID), 1)
        mask = (col_i // _HID == row_i).astype(jnp.float32)
        sblk_ref[...] = (s_tile * mask).astype(jnp.bfloat16)

    dt = dt_ref[...]
    w1f = w1f_ref[...]
    sblk = sblk_ref[...]
    c1 = (1.0 + a_ref[0, 0]) * 0.5

    pts = [
        jnp.dot(w1f, src_ref[j].T.astype(jnp.bfloat16),
                preferred_element_type=jnp.float32)
        for j in range(_U)
    ]
    pcat = jnp.concatenate(pts, axis=0)
    q = jnp.sum(pcat.reshape(_U, _HID, _N), axis=1)
    pcat16 = pcat.astype(jnp.bfloat16)
    hcat = jnp.dot(pcat16, dt, preferred_element_type=jnp.float32)
    tabs = jnp.abs(hcat).astype(jnp.bfloat16)
    o_nl = jnp.dot(sblk, tabs, preferred_element_type=jnp.float32)
    o_lin = jnp.dot(q.astype(jnp.bfloat16), dt,
                    preferred_element_type=jnp.float32)
    out_ref[...] = c1 * o_lin + o_nl


def kernel(src, s_SPD, W1, a, W2):
    spd = s_SPD.reshape(_E, _K)
    a2 = a.reshape(1, 1)
    out = pl.pallas_call(
        _edge_attn_body,
        grid=(_G,),
        in_specs=[
            pl.BlockSpec((_E, _K), lambda i: (0, 0)),
            pl.BlockSpec((_U, _N, _C), lambda i: (i, 0, 0)),
            pl.BlockSpec((_HID, _C), lambda i: (0, 0)),
            pl.BlockSpec((1, 1), lambda i: (0, 0)),
            pl.BlockSpec((1, _HID), lambda i: (0, 0)),
        ],
        out_specs=pl.BlockSpec((_U, _E), lambda i: (i, 0)),
        out_shape=jax.ShapeDtypeStruct((_B, _E), jnp.float32),
        scratch_shapes=[
            pltpu.VMEM((_N, _E), jnp.bfloat16),
            pltpu.VMEM((_HID, _C), jnp.bfloat16),
            pltpu.VMEM((_U, _U * _HID), jnp.bfloat16),
        ],
    )(spd, src, W1, a2, W2)
    return out.reshape(_B, _J, _J, 1)
